# baseline (speedup 1.0000x reference)
import functools

import jax
import jax.numpy as jnp
from jax.experimental import pallas as pl
from jax.experimental.pallas import tpu as pltpu

_BM = 2048


def _fused_body(x_hbm, w_ref, out_ref, x_vmem, wp_ref, acc_ref, sbuf, sems):
    i = pl.program_id(0)
    nprog = pl.num_programs(0)
    nb = nprog // 3
    hm = _BM // 2

    def _blk_copy(k):
        return pltpu.make_async_copy(
            x_hbm.at[pl.ds(k * _BM, _BM), :],
            x_vmem.at[pl.ds(k * _BM, _BM), :],
            sems.at[k],
        )

    @pl.when(i == 0)
    def _():
        for k in range(8):
            _blk_copy(k).start()
        w = w_ref[...]
        rinv = jax.lax.rsqrt(jnp.sum(w * w, axis=1, keepdims=True))
        wp_ref[...] = (w * rinv).astype(jnp.bfloat16)

    @pl.when(i < nb)
    def _():
        _blk_copy(i).wait()
        blk = x_vmem[pl.ds(i * _BM, _BM), :]
        psum = jnp.sum(blk * blk, axis=0, keepdims=True)

        @pl.when(i == 0)
        def _():
            acc_ref[...] = psum

        @pl.when(i > 0)
        def _():
            acc_ref[...] = acc_ref[...] + psum

        @pl.when(i == nb - 1)
        def _():
            acc_ref[...] = jax.lax.rsqrt(acc_ref[...])

    @pl.when(jnp.logical_and(i >= nb, (i - nb) % 2 == 0))
    def _():
        j = (i - nb) // 2
        xblk = (x_vmem[pl.ds(j * _BM, _BM), :] * acc_ref[...]).astype(jnp.bfloat16)
        beta = jax.lax.dot_general(
            xblk, wp_ref[...],
            dimension_numbers=(((1,), (1,)), ((), ())),
            preferred_element_type=jnp.float32,
        )
        e = jnp.exp(beta)
        sm = e * (1.0 / jnp.sum(e, axis=1, keepdims=True))
        out_ref[...] = sm[:hm, :]
        sbuf[...] = sm[hm:, :]

    @pl.when(jnp.logical_and(i >= nb, (i - nb) % 2 == 1))
    def _():
        out_ref[...] = sbuf[...]


@functools.partial(jax.jit, static_argnames=("interpret",))
def _snn_softmax(all_ts, W, interpret=False):
    B, TS = all_ts.shape
    N = W.shape[0]
    nb = B // _BM
    hm = _BM // 2
    out = pl.pallas_call(
        _fused_body,
        grid=(3 * nb,),
        in_specs=[
            pl.BlockSpec(memory_space=pl.ANY),
            pl.BlockSpec((N, TS), lambda i: (0, 0)),
        ],
        out_specs=pl.BlockSpec((hm, N), lambda i: (jnp.maximum(i - nb, 0), 0)),
        out_shape=jax.ShapeDtypeStruct((B, N), jnp.float32),
        scratch_shapes=[
            pltpu.VMEM((B, TS), jnp.float32),
            pltpu.VMEM((N, TS), jnp.bfloat16),
            pltpu.VMEM((1, TS), jnp.float32),
            pltpu.VMEM((hm, N), jnp.float32),
            pltpu.SemaphoreType.DMA((nb,)),
        ],
        interpret=interpret,
    )(all_ts, W)
    return out


def kernel(all_ts, W, cumhisto, clustering_flag):
    x = all_ts.reshape(all_ts.shape[0], -1)
    return _snn_softmax(x, W)


# speedup vs baseline: 1.2694x; 1.2694x over previous
import functools

import jax
import jax.numpy as jnp
from jax.experimental import pallas as pl
from jax.experimental.pallas import tpu as pltpu

_BM = 2048


def _fused_body(x_hbm, w_ref, out_ref, x_vmem, wp_ref, wr_ref, acc_ref, sems):
    i = pl.program_id(0)
    nb = pl.num_programs(0) // 2

    def _blk_copy(k):
        return pltpu.make_async_copy(
            x_hbm.at[pl.ds(k * _BM, _BM), :],
            x_vmem.at[pl.ds(k * _BM, _BM), :],
            sems.at[k],
        )

    @pl.when(i == 0)
    def _():
        for k in range(8):
            _blk_copy(k).start()
        w = w_ref[...]
        rinv = jax.lax.rsqrt(jnp.sum(w * w, axis=1, keepdims=True))
        wr_ref[...] = w * rinv

    @pl.when(i < nb)
    def _():
        _blk_copy(i).wait()
        blk = x_vmem[pl.ds(i * _BM, _BM), :]
        psum = jnp.sum(blk * blk, axis=0, keepdims=True)

        @pl.when(i == 0)
        def _():
            acc_ref[...] = psum

        @pl.when(i > 0)
        def _():
            acc_ref[...] = acc_ref[...] + psum

        @pl.when(i == nb - 1)
        def _():
            cinv = jax.lax.rsqrt(acc_ref[...])
            wp_ref[...] = (wr_ref[...] * cinv).astype(jnp.bfloat16)

    @pl.when(i >= nb)
    def _():
        j = i - nb
        xblk = x_vmem[pl.ds(j * _BM, _BM), :].astype(jnp.bfloat16)
        beta = jax.lax.dot_general(
            xblk, wp_ref[...],
            dimension_numbers=(((1,), (1,)), ((), ())),
            preferred_element_type=jnp.float32,
        )
        e = jnp.exp(beta)
        out_ref[...] = e * (1.0 / jnp.sum(e, axis=1, keepdims=True))


@functools.partial(jax.jit, static_argnames=("interpret",))
def _snn_softmax(all_ts, W, interpret=False):
    B, TS = all_ts.shape
    N = W.shape[0]
    nb = B // _BM
    out = pl.pallas_call(
        _fused_body,
        grid=(2 * nb,),
        in_specs=[
            pl.BlockSpec(memory_space=pl.ANY),
            pl.BlockSpec((N, TS), lambda i: (0, 0)),
        ],
        out_specs=pl.BlockSpec((_BM, N), lambda i: (jnp.maximum(i - nb, 0), 0)),
        out_shape=jax.ShapeDtypeStruct((B, N), jnp.float32),
        scratch_shapes=[
            pltpu.VMEM((B, TS), jnp.float32),
            pltpu.VMEM((N, TS), jnp.bfloat16),
            pltpu.VMEM((N, TS), jnp.float32),
            pltpu.VMEM((1, TS), jnp.float32),
            pltpu.SemaphoreType.DMA((8,)),
        ],
        interpret=interpret,
    )(all_ts, W)
    return out


def kernel(all_ts, W, cumhisto, clustering_flag):
    x = all_ts.reshape(all_ts.shape[0], -1)
    return _snn_softmax(x, W)
